# baseline (speedup 1.0000x reference)
import functools

import jax
import jax.numpy as jnp
from jax.experimental import pallas as pl

B = 1024
L = 200
D = 128
NNEG = 5
PAIRS = 1 + NNEG
OUT_ROWS = B * PAIRS

BLK = 128
GR = 512

_IDX_CACHE = []


def _sample_indices():
    if not _IDX_CACHE:
        import numpy as np
        with jax.ensure_compile_time_eval(), \
             jax.default_device(jax.devices("cpu")[0]):
            nkey = jax.random.key(42)
            r = jax.random.randint(nkey, (B, NNEG), 0, B - 1)
            neg = (jnp.arange(B)[:, None] + 1 + r) % B
            z2_idx = jnp.concatenate(
                [jnp.arange(B)[:, None], neg], axis=1).reshape(-1)
            _IDX_CACHE.append(np.asarray(z2_idx, dtype=np.int32))
    return _IDX_CACHE[0]


def _mean_kernel(x1_ref, x2_ref, z1_ref, m2_ref):
    inv_l = jnp.float32(1.0 / L)
    m1 = jnp.sum(x1_ref[...], axis=1) * inv_l
    m2 = jnp.sum(x2_ref[...], axis=1) * inv_l
    rep = jnp.broadcast_to(m1[:, None, :], (BLK, PAIRS, D))
    z1_ref[...] = rep.reshape(BLK * PAIRS, D)
    m2_ref[...] = m2


def _gather_kernel(idx_ref, m2_ref, z2_ref):
    idx = idx_ref[0]
    tbl = jax.lax.broadcasted_iota(jnp.int32, (B, GR), 0)
    oh = (tbl == idx).astype(jnp.float32)
    z2_ref[...] = jax.lax.dot_general(
        oh, m2_ref[...],
        dimension_numbers=(((0,), (0,)), ((), ())),
        preferred_element_type=jnp.float32,
    )


@functools.partial(jax.jit, static_argnames=())
def kernel(embeds, embeds_2, pids):
    del pids

    idx3 = jnp.asarray(_sample_indices()).reshape(OUT_ROWS // GR, 1, GR)

    z1_flat, m2 = pl.pallas_call(
        _mean_kernel,
        grid=(B // BLK,),
        in_specs=[
            pl.BlockSpec((BLK, L, D), lambda i: (i, 0, 0)),
            pl.BlockSpec((BLK, L, D), lambda i: (i, 0, 0)),
        ],
        out_specs=[
            pl.BlockSpec((BLK * PAIRS, D), lambda i: (i, 0)),
            pl.BlockSpec((BLK, D), lambda i: (i, 0)),
        ],
        out_shape=[
            jax.ShapeDtypeStruct((OUT_ROWS, D), jnp.float32),
            jax.ShapeDtypeStruct((B, D), jnp.float32),
        ],
    )(embeds, embeds_2)

    z2_flat = pl.pallas_call(
        _gather_kernel,
        grid=(OUT_ROWS // GR,),
        in_specs=[
            pl.BlockSpec((1, 1, GR), lambda i: (i, 0, 0)),
            pl.BlockSpec((B, D), lambda i: (0, 0)),
        ],
        out_specs=pl.BlockSpec((GR, D), lambda i: (i, 0)),
        out_shape=jax.ShapeDtypeStruct((OUT_ROWS, D), jnp.float32),
    )(idx3, m2)

    labels = jnp.tile(
        jnp.concatenate([jnp.ones((1,), jnp.float32), jnp.zeros((NNEG,), jnp.float32)]),
        B,
    )
    return (z1_flat[:, None, :], z2_flat[:, None, :], labels)


# speedup vs baseline: 1.1338x; 1.1338x over previous
import functools

import jax
import jax.numpy as jnp
from jax.experimental import pallas as pl
from jax.experimental.pallas import tpu as pltpu

B = 1024
L = 200
D = 128
NNEG = 5
PAIRS = 1 + NNEG
OUT_ROWS = B * PAIRS

BLK = 64
NSTEP = B // BLK
GR = OUT_ROWS // NSTEP

_IDX_CACHE = []


def _sample_indices():
    if not _IDX_CACHE:
        import numpy as np
        with jax.ensure_compile_time_eval(), \
             jax.default_device(jax.devices("cpu")[0]):
            nkey = jax.random.key(42)
            r = jax.random.randint(nkey, (B, NNEG), 0, B - 1)
            neg = (jnp.arange(B)[:, None] + 1 + r) % B
            z2_idx = jnp.concatenate(
                [jnp.arange(B)[:, None], neg], axis=1).reshape(-1)
            _IDX_CACHE.append(np.asarray(z2_idx, dtype=np.int32))
    return _IDX_CACHE[0]


def _fused_kernel(idx_ref, x2_ref, x1_ref, z1_ref, z2_ref, m2_ref):
    i = pl.program_id(0)
    inv_l = jnp.float32(1.0 / L)

    @pl.when(i < NSTEP)
    def _first_half():
        m2 = jnp.sum(x2_ref[...], axis=1) * inv_l
        m2_ref[pl.ds(i * BLK, BLK), :] = m2

    @pl.when(i >= NSTEP)
    def _second_half():
        g = i - NSTEP
        m1 = jnp.sum(x1_ref[...], axis=1) * inv_l
        rep = jnp.broadcast_to(m1[:, None, :], (BLK, PAIRS, D))
        z1_ref[...] = rep.reshape(BLK * PAIRS, D)
        idx = idx_ref[g]
        tbl = jax.lax.broadcasted_iota(jnp.int32, (B, GR), 0)
        oh = (tbl == idx).astype(jnp.float32)
        z2_ref[pl.ds(g * GR, GR), :] = jax.lax.dot_general(
            oh, m2_ref[...],
            dimension_numbers=(((0,), (0,)), ((), ())),
            preferred_element_type=jnp.float32,
        )


@functools.partial(jax.jit, static_argnames=())
def kernel(embeds, embeds_2, pids):
    del pids

    idx3 = jnp.asarray(_sample_indices()).reshape(NSTEP, 1, GR)

    z1_flat, z2_flat = pl.pallas_call(
        _fused_kernel,
        grid=(2 * NSTEP,),
        in_specs=[
            pl.BlockSpec((NSTEP, 1, GR), lambda i: (0, 0, 0)),
            pl.BlockSpec((BLK, L, D),
                         lambda i: (jnp.minimum(i, NSTEP - 1), 0, 0)),
            pl.BlockSpec((BLK, L, D),
                         lambda i: (jnp.maximum(i - NSTEP, 0), 0, 0)),
        ],
        out_specs=[
            pl.BlockSpec((BLK * PAIRS, D), lambda i: (jnp.maximum(i - NSTEP, 0), 0)),
            pl.BlockSpec((OUT_ROWS, D), lambda i: (0, 0)),
        ],
        out_shape=[
            jax.ShapeDtypeStruct((OUT_ROWS, D), jnp.float32),
            jax.ShapeDtypeStruct((OUT_ROWS, D), jnp.float32),
        ],
        scratch_shapes=[pltpu.VMEM((B, D), jnp.float32)],
    )(idx3, embeds_2, embeds)

    labels = jnp.tile(
        jnp.concatenate([jnp.ones((1,), jnp.float32), jnp.zeros((NNEG,), jnp.float32)]),
        B,
    )
    return (z1_flat[:, None, :], z2_flat[:, None, :], labels)
